# Initial kernel scaffold; baseline (speedup 1.0000x reference)
import jax
import jax.numpy as jnp
from jax.experimental import pallas as pl


def kernel(x, edge_index, params):
    raise NotImplementedError("write your pallas kernel here")


# speedup vs baseline: 14.4764x; 14.4764x over previous
import functools

import jax
import jax.numpy as jnp
from jax import lax
from jax.experimental import pallas as pl
from jax.experimental.pallas import tpu as pltpu
from jax.experimental.pallas import tpu_sc as plsc

_N = 10000
_E = 320000
_IN = 128
_HID = 64
_OUT = 32

_NC = 2
_NS = 16
_NW = _NC * _NS
_CB = 125
_ROWS = _E // _CB
_RPW = _ROWS // _NW
_KI = 16
_NCHUNK = _RPW // _KI
_ZR = 624
_ZTAIL = _N - _NS * _ZR


def _sc_mesh():
    return plsc.VectorSubcoreMesh(
        core_axis_name="c", subcore_axis_name="s",
        num_cores=_NC, num_subcores=_NS)


_NBUF = 4


def _sc_segsum(h, srcr, dstr, zrows):
    @functools.partial(
        pl.kernel,
        out_type=jax.ShapeDtypeStruct((_NC, _N, _HID), jnp.float32),
        mesh=_sc_mesh(),
        scratch_types=[
            pltpu.VMEM((_RPW, _CB), jnp.int32),
            pltpu.VMEM((_RPW, _CB), jnp.int32),
            pltpu.VMEM((_NBUF, _CB, _HID), jnp.float32),
            pltpu.VMEM_SHARED((_N, _HID), jnp.float32),
        ] + [pltpu.SemaphoreType.DMA] * (2 * _NBUF),
        compiler_params=pltpu.CompilerParams(use_tc_tiling_on_sc=False),
    )
    def run(h_hbm, src_hbm, dst_hbm, z_hbm, out_hbm,
            src_v, dst_v, rows_v, acc_sh, *sems):
        gsem = sems[:_NBUF]
        ssem = sems[_NBUF:]
        c = lax.axis_index("c")
        s = lax.axis_index("s")
        w = c * _NS + s
        pltpu.sync_copy(z_hbm, acc_sh.at[pl.ds(s * _ZR, _ZR)])

        @pl.when(s == 0)
        def _ztail():
            pltpu.sync_copy(z_hbm.at[pl.ds(0, _ZTAIL)],
                            acc_sh.at[pl.ds(_NS * _ZR, _ZTAIL)])

        plsc.subcore_barrier()

        base = w * _RPW
        pltpu.sync_copy(src_hbm.at[pl.ds(base, _RPW)], src_v)
        pltpu.sync_copy(dst_hbm.at[pl.ds(base, _RPW)], dst_v)

        def g_desc(b, j):
            return pltpu.make_async_copy(
                h_hbm.at[src_v.at[j]], rows_v.at[b], gsem[b])

        def s_desc(b, j):
            return pltpu.make_async_copy(
                rows_v.at[b], acc_sh.at[dst_v.at[j]], ssem[b])

        for b in range(_NBUF):
            g_desc(b, b).start()

        def step(k, carry):
            j0 = k * _NBUF
            for b in range(_NBUF):
                g_desc(b, j0 + b).wait()
                pltpu.async_copy(rows_v.at[b], acc_sh.at[dst_v.at[j0 + b]],
                                 ssem[b], add=True)
            for b in range(_NBUF):
                s_desc(b, j0 + b).wait()
                g_desc(b, j0 + _NBUF + b).start()
            return carry

        lax.fori_loop(0, _RPW // _NBUF - 1, step, 0)

        j0 = _RPW - _NBUF
        for b in range(_NBUF):
            g_desc(b, j0 + b).wait()
            pltpu.async_copy(rows_v.at[b], acc_sh.at[dst_v.at[j0 + b]],
                             ssem[b], add=True)
        for b in range(_NBUF):
            s_desc(b, j0 + b).wait()

        plsc.subcore_barrier()
        pltpu.sync_copy(acc_sh.at[pl.ds(s * _ZR, _ZR)],
                        out_hbm.at[c, pl.ds(s * _ZR, _ZR)])

        @pl.when(s == 0)
        def _otail():
            pltpu.sync_copy(acc_sh.at[pl.ds(_NS * _ZR, _ZTAIL)],
                            out_hbm.at[c, pl.ds(_NS * _ZR, _ZTAIL)])

    return run(h, srcr, dstr, zrows)


def _sc_deg(dstr, ones_rows, zrows):
    @functools.partial(
        pl.kernel,
        out_type=jax.ShapeDtypeStruct((_NC, _N, 8), jnp.float32),
        mesh=_sc_mesh(),
        scratch_types=[
            pltpu.VMEM((_KI, _CB), jnp.int32),
            pltpu.VMEM((_CB, 8), jnp.float32),
            pltpu.VMEM_SHARED((_N, 8), jnp.float32),
        ],
        compiler_params=pltpu.CompilerParams(use_tc_tiling_on_sc=False),
    )
    def run(dst_hbm, ones_hbm, z_hbm, out_hbm, dst_v, ones_v, acc_sh):
        c = lax.axis_index("c")
        s = lax.axis_index("s")
        w = c * _NS + s
        pltpu.sync_copy(ones_hbm, ones_v)
        pltpu.sync_copy(z_hbm, acc_sh.at[pl.ds(s * _ZR, _ZR)])

        @pl.when(s == 0)
        def _ztail():
            pltpu.sync_copy(z_hbm.at[pl.ds(0, _ZTAIL)],
                            acc_sh.at[pl.ds(_NS * _ZR, _ZTAIL)])

        plsc.subcore_barrier()
        base = w * _RPW

        def chunk(k, carry):
            r0 = base + k * _KI
            pltpu.sync_copy(dst_hbm.at[pl.ds(r0, _KI)], dst_v)

            def row(j, carry2):
                pltpu.sync_copy(ones_v, acc_sh.at[dst_v.at[j]], add=True)
                return carry2

            return lax.fori_loop(0, _KI, row, carry)

        lax.fori_loop(0, _NCHUNK, chunk, 0)
        plsc.subcore_barrier()
        pltpu.sync_copy(acc_sh.at[pl.ds(s * _ZR, _ZR)],
                        out_hbm.at[c, pl.ds(s * _ZR, _ZR)])

        @pl.when(s == 0)
        def _otail():
            pltpu.sync_copy(acc_sh.at[pl.ds(_NS * _ZR, _ZTAIL)],
                            out_hbm.at[c, pl.ds(_NS * _ZR, _ZTAIL)])

    return run(dstr, ones_rows, zrows)


_R = 1000


def _tc_encoder(x, wt, b):
    def body(x_ref, w_ref, b_ref, o_ref):
        o_ref[...] = jnp.dot(x_ref[...], w_ref[...],
                             preferred_element_type=jnp.float32) + b_ref[...]

    return pl.pallas_call(
        body,
        grid=(_N // _R,),
        in_specs=[
            pl.BlockSpec((_R, _IN), lambda i: (i, 0)),
            pl.BlockSpec((_IN, _HID), lambda i: (0, 0)),
            pl.BlockSpec((1, _HID), lambda i: (0, 0)),
        ],
        out_specs=pl.BlockSpec((_R, _HID), lambda i: (i, 0)),
        out_shape=jax.ShapeDtypeStruct((_N, _HID), jnp.float32),
    )(x, wt, b)


def _tc_layer(part, degp, h, wlt, wrt, scale, bias):
    def body(p_ref, d_ref, h_ref, wl_ref, wr_ref, s_ref, b_ref, o_ref):
        deg = d_ref[0, :, 0:1] + d_ref[1, :, 0:1]
        inv = 1.0 / jnp.maximum(deg, 1.0)
        aggr = (p_ref[0] + p_ref[1]) * inv
        y = (jnp.dot(aggr, wl_ref[...], preferred_element_type=jnp.float32)
             + jnp.dot(h_ref[...], wr_ref[...],
                       preferred_element_type=jnp.float32))
        o_ref[...] = jnp.maximum(y * s_ref[...] + b_ref[...], 0.0)

    return pl.pallas_call(
        body,
        grid=(_N // _R,),
        in_specs=[
            pl.BlockSpec((_NC, _R, _HID), lambda i: (0, i, 0)),
            pl.BlockSpec((_NC, _R, 8), lambda i: (0, i, 0)),
            pl.BlockSpec((_R, _HID), lambda i: (i, 0)),
            pl.BlockSpec((_HID, _HID), lambda i: (0, 0)),
            pl.BlockSpec((_HID, _HID), lambda i: (0, 0)),
            pl.BlockSpec((1, _HID), lambda i: (0, 0)),
            pl.BlockSpec((1, _HID), lambda i: (0, 0)),
        ],
        out_specs=pl.BlockSpec((_R, _HID), lambda i: (i, 0)),
        out_shape=jax.ShapeDtypeStruct((_N, _HID), jnp.float32),
    )(part, degp, h, wlt, wrt, scale, bias)


def _tc_final(part, degp, h, wlt, wrt, scale, bias, heads):
    (ew1t, eb1, ew2t, eb2, mw1t, mb1, mw2t, mb2, fw1t, fb1, fw2t, fb2) = heads
    steps = _N // _R

    def body(p_ref, d_ref, h_ref, wl_ref, wr_ref, s_ref, b_ref,
             ew1_ref, eb1_ref, ew2_ref, eb2_ref,
             mw1_ref, mb1_ref, mw2_ref, mb2_ref,
             fw1_ref, fb1_ref, fw2_ref, fb2_ref,
             o_ref, g_ref, eth_ref, man_ref, mf_ref):
        i = pl.program_id(0)
        deg = d_ref[0, :, 0:1] + d_ref[1, :, 0:1]
        inv = 1.0 / jnp.maximum(deg, 1.0)
        aggr = (p_ref[0] + p_ref[1]) * inv
        y = (jnp.dot(aggr, wl_ref[...], preferred_element_type=jnp.float32)
             + jnp.dot(h_ref[...], wr_ref[...],
                       preferred_element_type=jnp.float32))
        y = y * s_ref[...] + b_ref[...]
        o_ref[...] = y
        blk = jnp.sum(y, axis=0, keepdims=True)

        @pl.when(i == 0)
        def _init():
            g_ref[...] = blk

        @pl.when(i > 0)
        def _acc():
            g_ref[...] = g_ref[...] + blk

        @pl.when(i == steps - 1)
        def _heads():
            g = g_ref[...] / float(_N)
            g_ref[...] = g
            e1 = jnp.maximum(
                jnp.dot(g, ew1_ref[...], preferred_element_type=jnp.float32)
                + eb1_ref[...], 0.0)
            eth_ref[...] = jax.nn.sigmoid(
                jnp.dot(e1, ew2_ref[...], preferred_element_type=jnp.float32)
                + eb2_ref[...])
            m1 = jnp.maximum(
                jnp.dot(g, mw1_ref[...], preferred_element_type=jnp.float32)
                + mb1_ref[...], 0.0)
            man_ref[...] = jax.nn.sigmoid(
                jnp.dot(m1, mw2_ref[...], preferred_element_type=jnp.float32)
                + mb2_ref[...])
            f1 = jnp.maximum(
                jnp.dot(g, fw1_ref[...], preferred_element_type=jnp.float32)
                + fb1_ref[...], 0.0)
            mf_ref[...] = (
                jnp.dot(f1, fw2_ref[...], preferred_element_type=jnp.float32)
                + fb2_ref[...])

    full = lambda shape: pl.BlockSpec(shape, lambda i: tuple(0 for _ in shape))
    return pl.pallas_call(
        body,
        grid=(steps,),
        in_specs=[
            pl.BlockSpec((_NC, _R, _HID), lambda i: (0, i, 0)),
            pl.BlockSpec((_NC, _R, 8), lambda i: (0, i, 0)),
            pl.BlockSpec((_R, _HID), lambda i: (i, 0)),
            full((_HID, _OUT)),
            full((_HID, _OUT)),
            full((1, _OUT)),
            full((1, _OUT)),
            full((_OUT, 16)), full((1, 16)), full((16, 1)), full((1, 1)),
            full((_OUT, 16)), full((1, 16)), full((16, 1)), full((1, 1)),
            full((_OUT, 16)), full((1, 16)), full((16, 6)), full((1, 6)),
        ],
        out_specs=[
            pl.BlockSpec((_R, _OUT), lambda i: (i, 0)),
            full((1, _OUT)),
            full((1, 1)),
            full((1, 1)),
            full((1, 6)),
        ],
        out_shape=[
            jax.ShapeDtypeStruct((_N, _OUT), jnp.float32),
            jax.ShapeDtypeStruct((1, _OUT), jnp.float32),
            jax.ShapeDtypeStruct((1, 1), jnp.float32),
            jax.ShapeDtypeStruct((1, 1), jnp.float32),
            jax.ShapeDtypeStruct((1, 6), jnp.float32),
        ],
    )(part, degp, h, wlt, wrt, scale, bias,
      ew1t, eb1, ew2t, eb2, mw1t, mb1, mw2t, mb2, fw1t, fb1, fw2t, fb2)


def kernel(x, edge_index, params):
    p = params
    src = edge_index[0].reshape(_ROWS, _CB)
    dst = edge_index[1].reshape(_ROWS, _CB)
    zrows = jnp.zeros((_ZR, _HID), jnp.float32)
    zrows8 = jnp.zeros((_ZR, 8), jnp.float32)
    ones_rows = jnp.ones((_CB, 8), jnp.float32)

    degp = _sc_deg(dst, ones_rows, zrows8)
    h = _tc_encoder(x, p['enc_W'].T, p['enc_b'].reshape(1, -1))

    for i in range(3):
        lp = p['sage'][i]
        bn = p['bn'][i]
        scale = (bn['gamma'] / jnp.sqrt(bn['var'] + 1e-5)).reshape(1, -1)
        bias = ((lp['bl'] - bn['mean']).reshape(1, -1) * scale
                + bn['beta'].reshape(1, -1))
        part = _sc_segsum(h, src, dst, zrows)
        if i < 2:
            h = _tc_layer(part, degp, h, lp['Wl'].T, lp['Wr'].T, scale, bias)
        else:
            heads = (
                p['eth_W1'].T, p['eth_b1'].reshape(1, -1),
                p['eth_W2'].T, p['eth_b2'].reshape(1, -1),
                p['man_W1'].T, p['man_b1'].reshape(1, -1),
                p['man_W2'].T, p['man_b2'].reshape(1, -1),
                p['mf_W1'].T, p['mf_b1'].reshape(1, -1),
                p['mf_W2'].T, p['mf_b2'].reshape(1, -1),
            )
            node_emb, g, eth, man, mf = _tc_final(
                part, degp, h, lp['Wl'].T, lp['Wr'].T, scale, bias, heads)
    return (node_emb, g, eth, man, mf)


# speedup vs baseline: 14.6621x; 1.0128x over previous
import functools

import jax
import jax.numpy as jnp
from jax import lax
from jax.experimental import pallas as pl
from jax.experimental.pallas import tpu as pltpu
from jax.experimental.pallas import tpu_sc as plsc

_N = 10000
_E = 320000
_IN = 128
_HID = 64
_OUT = 32

_NC = 2
_NS = 16
_NW = _NC * _NS
_CB = 125
_ROWS = _E // _CB
_RPW = _ROWS // _NW
_ZR = 624
_ZTAIL = _N - _NS * _ZR


def _sc_mesh():
    return plsc.VectorSubcoreMesh(
        core_axis_name="c", subcore_axis_name="s",
        num_cores=_NC, num_subcores=_NS)


_NBUF = 4


def _sc_segsum(h, srcr, dstr, zrows):
    @functools.partial(
        pl.kernel,
        out_type=jax.ShapeDtypeStruct((_NC, _N, _HID), jnp.float32),
        mesh=_sc_mesh(),
        scratch_types=[
            pltpu.VMEM((_RPW, _CB), jnp.int32),
            pltpu.VMEM((_RPW, _CB), jnp.int32),
            pltpu.VMEM((_NBUF, _CB, _HID), jnp.float32),
            pltpu.VMEM_SHARED((_N, _HID), jnp.float32),
        ] + [pltpu.SemaphoreType.DMA] * (2 * _NBUF),
        compiler_params=pltpu.CompilerParams(use_tc_tiling_on_sc=False),
    )
    def run(h_hbm, src_hbm, dst_hbm, z_hbm, out_hbm,
            src_v, dst_v, rows_v, acc_sh, *sems):
        gsem = sems[:_NBUF]
        ssem = sems[_NBUF:]
        c = lax.axis_index("c")
        s = lax.axis_index("s")
        w = c * _NS + s
        pltpu.sync_copy(z_hbm, acc_sh.at[pl.ds(s * _ZR, _ZR)])

        @pl.when(s == 0)
        def _ztail():
            pltpu.sync_copy(z_hbm.at[pl.ds(0, _ZTAIL)],
                            acc_sh.at[pl.ds(_NS * _ZR, _ZTAIL)])

        plsc.subcore_barrier()

        base = w * _RPW
        pltpu.sync_copy(src_hbm.at[pl.ds(base, _RPW)], src_v)
        pltpu.sync_copy(dst_hbm.at[pl.ds(base, _RPW)], dst_v)

        def g_desc(b, j):
            return pltpu.make_async_copy(
                h_hbm.at[src_v.at[j]], rows_v.at[b], gsem[b])

        def s_desc(b, j):
            return pltpu.make_async_copy(
                rows_v.at[b], acc_sh.at[dst_v.at[j]], ssem[b])

        for b in range(_NBUF):
            g_desc(b, b).start()

        def step(k, carry):
            j0 = k * _NBUF
            for b in range(_NBUF):
                g_desc(b, j0 + b).wait()
                pltpu.async_copy(rows_v.at[b], acc_sh.at[dst_v.at[j0 + b]],
                                 ssem[b], add=True)
            for b in range(_NBUF):
                s_desc(b, j0 + b).wait()
                g_desc(b, j0 + _NBUF + b).start()
            return carry

        lax.fori_loop(0, _RPW // _NBUF - 1, step, 0)

        j0 = _RPW - _NBUF
        for b in range(_NBUF):
            g_desc(b, j0 + b).wait()
            pltpu.async_copy(rows_v.at[b], acc_sh.at[dst_v.at[j0 + b]],
                             ssem[b], add=True)
        for b in range(_NBUF):
            s_desc(b, j0 + b).wait()

        plsc.subcore_barrier()
        pltpu.sync_copy(acc_sh.at[pl.ds(s * _ZR, _ZR)],
                        out_hbm.at[c, pl.ds(s * _ZR, _ZR)])

        @pl.when(s == 0)
        def _otail():
            pltpu.sync_copy(acc_sh.at[pl.ds(_NS * _ZR, _ZTAIL)],
                            out_hbm.at[c, pl.ds(_NS * _ZR, _ZTAIL)])

    return run(h, srcr, dstr, zrows)


def _sc_segsum_deg(h, srcr, dstr, zrows, zrows8, ones_rows):
    @functools.partial(
        pl.kernel,
        out_type=(jax.ShapeDtypeStruct((_NC, _N, _HID), jnp.float32),
                  jax.ShapeDtypeStruct((_NC, _N, 8), jnp.float32)),
        mesh=_sc_mesh(),
        scratch_types=[
            pltpu.VMEM((_RPW, _CB), jnp.int32),
            pltpu.VMEM((_RPW, _CB), jnp.int32),
            pltpu.VMEM((_NBUF, _CB, _HID), jnp.float32),
            pltpu.VMEM((_CB, 8), jnp.float32),
            pltpu.VMEM_SHARED((_N, _HID), jnp.float32),
            pltpu.VMEM_SHARED((_N, 8), jnp.float32),
        ] + [pltpu.SemaphoreType.DMA] * (2 * _NBUF),
        compiler_params=pltpu.CompilerParams(use_tc_tiling_on_sc=False),
    )
    def run(h_hbm, src_hbm, dst_hbm, z_hbm, z8_hbm, ones_hbm,
            out_hbm, deg_hbm, src_v, dst_v, rows_v, ones_v,
            acc_sh, dacc_sh, *sems):
        gsem = sems[:_NBUF]
        ssem = sems[_NBUF:]
        c = lax.axis_index("c")
        s = lax.axis_index("s")
        w = c * _NS + s
        pltpu.sync_copy(ones_hbm, ones_v)
        pltpu.sync_copy(z_hbm, acc_sh.at[pl.ds(s * _ZR, _ZR)])
        pltpu.sync_copy(z8_hbm, dacc_sh.at[pl.ds(s * _ZR, _ZR)])

        @pl.when(s == 0)
        def _ztail():
            pltpu.sync_copy(z_hbm.at[pl.ds(0, _ZTAIL)],
                            acc_sh.at[pl.ds(_NS * _ZR, _ZTAIL)])
            pltpu.sync_copy(z8_hbm.at[pl.ds(0, _ZTAIL)],
                            dacc_sh.at[pl.ds(_NS * _ZR, _ZTAIL)])

        plsc.subcore_barrier()

        base = w * _RPW
        pltpu.sync_copy(src_hbm.at[pl.ds(base, _RPW)], src_v)
        pltpu.sync_copy(dst_hbm.at[pl.ds(base, _RPW)], dst_v)

        def g_desc(b, j):
            return pltpu.make_async_copy(
                h_hbm.at[src_v.at[j]], rows_v.at[b], gsem[b])

        def s_desc(b, j):
            return pltpu.make_async_copy(
                rows_v.at[b], acc_sh.at[dst_v.at[j]], ssem[b])

        for b in range(_NBUF):
            g_desc(b, b).start()

        def step(k, carry):
            j0 = k * _NBUF
            for b in range(_NBUF):
                g_desc(b, j0 + b).wait()
                pltpu.async_copy(rows_v.at[b], acc_sh.at[dst_v.at[j0 + b]],
                                 ssem[b], add=True)
            for b in range(_NBUF):
                s_desc(b, j0 + b).wait()
                g_desc(b, j0 + _NBUF + b).start()
            return carry

        lax.fori_loop(0, _RPW // _NBUF - 1, step, 0)

        j0 = _RPW - _NBUF
        for b in range(_NBUF):
            g_desc(b, j0 + b).wait()
            pltpu.async_copy(rows_v.at[b], acc_sh.at[dst_v.at[j0 + b]],
                             ssem[b], add=True)
        for b in range(_NBUF):
            s_desc(b, j0 + b).wait()

        def dphase(j, carry):
            pltpu.sync_copy(ones_v, dacc_sh.at[dst_v.at[j]], add=True)
            return carry

        lax.fori_loop(0, _RPW, dphase, 0)

        plsc.subcore_barrier()
        pltpu.sync_copy(acc_sh.at[pl.ds(s * _ZR, _ZR)],
                        out_hbm.at[c, pl.ds(s * _ZR, _ZR)])
        pltpu.sync_copy(dacc_sh.at[pl.ds(s * _ZR, _ZR)],
                        deg_hbm.at[c, pl.ds(s * _ZR, _ZR)])

        @pl.when(s == 0)
        def _otail():
            pltpu.sync_copy(acc_sh.at[pl.ds(_NS * _ZR, _ZTAIL)],
                            out_hbm.at[c, pl.ds(_NS * _ZR, _ZTAIL)])
            pltpu.sync_copy(dacc_sh.at[pl.ds(_NS * _ZR, _ZTAIL)],
                            deg_hbm.at[c, pl.ds(_NS * _ZR, _ZTAIL)])

    return run(h, srcr, dstr, zrows, zrows8, ones_rows)


_R = 1000


def _tc_encoder(x, wt, b):
    def body(x_ref, w_ref, b_ref, o_ref):
        o_ref[...] = jnp.dot(x_ref[...], w_ref[...],
                             preferred_element_type=jnp.float32) + b_ref[...]

    return pl.pallas_call(
        body,
        grid=(_N // _R,),
        in_specs=[
            pl.BlockSpec((_R, _IN), lambda i: (i, 0)),
            pl.BlockSpec((_IN, _HID), lambda i: (0, 0)),
            pl.BlockSpec((1, _HID), lambda i: (0, 0)),
        ],
        out_specs=pl.BlockSpec((_R, _HID), lambda i: (i, 0)),
        out_shape=jax.ShapeDtypeStruct((_N, _HID), jnp.float32),
    )(x, wt, b)


def _tc_layer(part, degp, h, wlt, wrt, scale, bias):
    def body(p_ref, d_ref, h_ref, wl_ref, wr_ref, s_ref, b_ref, o_ref):
        deg = d_ref[0, :, 0:1] + d_ref[1, :, 0:1]
        inv = 1.0 / jnp.maximum(deg, 1.0)
        aggr = (p_ref[0] + p_ref[1]) * inv
        y = (jnp.dot(aggr, wl_ref[...], preferred_element_type=jnp.float32)
             + jnp.dot(h_ref[...], wr_ref[...],
                       preferred_element_type=jnp.float32))
        o_ref[...] = jnp.maximum(y * s_ref[...] + b_ref[...], 0.0)

    return pl.pallas_call(
        body,
        grid=(_N // _R,),
        in_specs=[
            pl.BlockSpec((_NC, _R, _HID), lambda i: (0, i, 0)),
            pl.BlockSpec((_NC, _R, 8), lambda i: (0, i, 0)),
            pl.BlockSpec((_R, _HID), lambda i: (i, 0)),
            pl.BlockSpec((_HID, _HID), lambda i: (0, 0)),
            pl.BlockSpec((_HID, _HID), lambda i: (0, 0)),
            pl.BlockSpec((1, _HID), lambda i: (0, 0)),
            pl.BlockSpec((1, _HID), lambda i: (0, 0)),
        ],
        out_specs=pl.BlockSpec((_R, _HID), lambda i: (i, 0)),
        out_shape=jax.ShapeDtypeStruct((_N, _HID), jnp.float32),
    )(part, degp, h, wlt, wrt, scale, bias)


def _tc_final(part, degp, h, wlt, wrt, scale, bias, heads):
    (ew1t, eb1, ew2t, eb2, mw1t, mb1, mw2t, mb2, fw1t, fb1, fw2t, fb2) = heads
    steps = _N // _R

    def body(p_ref, d_ref, h_ref, wl_ref, wr_ref, s_ref, b_ref,
             ew1_ref, eb1_ref, ew2_ref, eb2_ref,
             mw1_ref, mb1_ref, mw2_ref, mb2_ref,
             fw1_ref, fb1_ref, fw2_ref, fb2_ref,
             o_ref, g_ref, eth_ref, man_ref, mf_ref):
        i = pl.program_id(0)
        deg = d_ref[0, :, 0:1] + d_ref[1, :, 0:1]
        inv = 1.0 / jnp.maximum(deg, 1.0)
        aggr = (p_ref[0] + p_ref[1]) * inv
        y = (jnp.dot(aggr, wl_ref[...], preferred_element_type=jnp.float32)
             + jnp.dot(h_ref[...], wr_ref[...],
                       preferred_element_type=jnp.float32))
        y = y * s_ref[...] + b_ref[...]
        o_ref[...] = y
        blk = jnp.sum(y, axis=0, keepdims=True)

        @pl.when(i == 0)
        def _init():
            g_ref[...] = blk

        @pl.when(i > 0)
        def _acc():
            g_ref[...] = g_ref[...] + blk

        @pl.when(i == steps - 1)
        def _heads():
            g = g_ref[...] / float(_N)
            g_ref[...] = g
            e1 = jnp.maximum(
                jnp.dot(g, ew1_ref[...], preferred_element_type=jnp.float32)
                + eb1_ref[...], 0.0)
            eth_ref[...] = jax.nn.sigmoid(
                jnp.dot(e1, ew2_ref[...], preferred_element_type=jnp.float32)
                + eb2_ref[...])
            m1 = jnp.maximum(
                jnp.dot(g, mw1_ref[...], preferred_element_type=jnp.float32)
                + mb1_ref[...], 0.0)
            man_ref[...] = jax.nn.sigmoid(
                jnp.dot(m1, mw2_ref[...], preferred_element_type=jnp.float32)
                + mb2_ref[...])
            f1 = jnp.maximum(
                jnp.dot(g, fw1_ref[...], preferred_element_type=jnp.float32)
                + fb1_ref[...], 0.0)
            mf_ref[...] = (
                jnp.dot(f1, fw2_ref[...], preferred_element_type=jnp.float32)
                + fb2_ref[...])

    full = lambda shape: pl.BlockSpec(shape, lambda i: tuple(0 for _ in shape))
    return pl.pallas_call(
        body,
        grid=(steps,),
        in_specs=[
            pl.BlockSpec((_NC, _R, _HID), lambda i: (0, i, 0)),
            pl.BlockSpec((_NC, _R, 8), lambda i: (0, i, 0)),
            pl.BlockSpec((_R, _HID), lambda i: (i, 0)),
            full((_HID, _OUT)),
            full((_HID, _OUT)),
            full((1, _OUT)),
            full((1, _OUT)),
            full((_OUT, 16)), full((1, 16)), full((16, 1)), full((1, 1)),
            full((_OUT, 16)), full((1, 16)), full((16, 1)), full((1, 1)),
            full((_OUT, 16)), full((1, 16)), full((16, 6)), full((1, 6)),
        ],
        out_specs=[
            pl.BlockSpec((_R, _OUT), lambda i: (i, 0)),
            full((1, _OUT)),
            full((1, 1)),
            full((1, 1)),
            full((1, 6)),
        ],
        out_shape=[
            jax.ShapeDtypeStruct((_N, _OUT), jnp.float32),
            jax.ShapeDtypeStruct((1, _OUT), jnp.float32),
            jax.ShapeDtypeStruct((1, 1), jnp.float32),
            jax.ShapeDtypeStruct((1, 1), jnp.float32),
            jax.ShapeDtypeStruct((1, 6), jnp.float32),
        ],
    )(part, degp, h, wlt, wrt, scale, bias,
      ew1t, eb1, ew2t, eb2, mw1t, mb1, mw2t, mb2, fw1t, fb1, fw2t, fb2)


def kernel(x, edge_index, params):
    p = params
    src = edge_index[0].reshape(_ROWS, _CB)
    dst = edge_index[1].reshape(_ROWS, _CB)
    zrows = jnp.zeros((_ZR, _HID), jnp.float32)
    zrows8 = jnp.zeros((_ZR, 8), jnp.float32)
    ones_rows = jnp.ones((_CB, 8), jnp.float32)

    h = _tc_encoder(x, p['enc_W'].T, p['enc_b'].reshape(1, -1))

    for i in range(3):
        lp = p['sage'][i]
        bn = p['bn'][i]
        scale = (bn['gamma'] / jnp.sqrt(bn['var'] + 1e-5)).reshape(1, -1)
        bias = ((lp['bl'] - bn['mean']).reshape(1, -1) * scale
                + bn['beta'].reshape(1, -1))
        if i == 0:
            part, degp = _sc_segsum_deg(h, src, dst, zrows, zrows8, ones_rows)
        else:
            part = _sc_segsum(h, src, dst, zrows)
        if i < 2:
            h = _tc_layer(part, degp, h, lp['Wl'].T, lp['Wr'].T, scale, bias)
        else:
            heads = (
                p['eth_W1'].T, p['eth_b1'].reshape(1, -1),
                p['eth_W2'].T, p['eth_b2'].reshape(1, -1),
                p['man_W1'].T, p['man_b1'].reshape(1, -1),
                p['man_W2'].T, p['man_b2'].reshape(1, -1),
                p['mf_W1'].T, p['mf_b1'].reshape(1, -1),
                p['mf_W2'].T, p['mf_b2'].reshape(1, -1),
            )
            node_emb, g, eth, man, mf = _tc_final(
                part, degp, h, lp['Wl'].T, lp['Wr'].T, scale, bias, heads)
    return (node_emb, g, eth, man, mf)


# speedup vs baseline: 16.4337x; 1.1208x over previous
import functools

import jax
import jax.numpy as jnp
from jax import lax
from jax.experimental import pallas as pl
from jax.experimental.pallas import tpu as pltpu
from jax.experimental.pallas import tpu_sc as plsc

_N = 10000
_E = 320000
_IN = 128
_HID = 64
_OUT = 32

_NC = 2
_NS = 16
_NW = _NC * _NS
_CB = 128
_ROWS = _E // _CB
_RPW = _ROWS // _NW
_RTAIL = _ROWS - _RPW * _NW
_ZR = 624
_ZTAIL = _N - _NS * _ZR
_NBUF = 3


def _sc_mesh():
    return plsc.VectorSubcoreMesh(
        core_axis_name="c", subcore_axis_name="s",
        num_cores=_NC, num_subcores=_NS)


def _segsum_body(h_hbm, src_hbm, dst_hbm, out_hbm,
                 src_v, dst_v, rows_v, acc_sh, gsem, ssem, c, s):
    w = c * _NS + s
    base = w * _RPW
    pltpu.sync_copy(src_hbm.at[pl.ds(base, _RPW)],
                    src_v.at[pl.ds(0, _RPW)])
    pltpu.sync_copy(dst_hbm.at[pl.ds(base, _RPW)],
                    dst_v.at[pl.ds(0, _RPW)])

    @pl.when(w < _RTAIL)
    def _tail_idx():
        pltpu.sync_copy(src_hbm.at[pl.ds(_RPW * _NW + w, 1)],
                        src_v.at[pl.ds(_RPW, 1)])
        pltpu.sync_copy(dst_hbm.at[pl.ds(_RPW * _NW + w, 1)],
                        dst_v.at[pl.ds(_RPW, 1)])

    def g_desc(b, j):
        return pltpu.make_async_copy(
            h_hbm.at[src_v.at[j]], rows_v.at[b], gsem[b])

    def s_desc(b, j):
        return pltpu.make_async_copy(
            rows_v.at[b], acc_sh.at[dst_v.at[j]], ssem[b])

    for b in range(_NBUF):
        g_desc(b, b).start()

    def step(k, carry):
        j0 = k * _NBUF
        for b in range(_NBUF):
            g_desc(b, j0 + b).wait()
            pltpu.async_copy(rows_v.at[b], acc_sh.at[dst_v.at[j0 + b]],
                             ssem[b], add=True)
        for b in range(_NBUF):
            s_desc(b, j0 + b).wait()
            g_desc(b, j0 + _NBUF + b).start()
        return carry

    lax.fori_loop(0, _RPW // _NBUF - 1, step, 0)

    j0 = _RPW - _NBUF
    for b in range(_NBUF):
        g_desc(b, j0 + b).wait()
        pltpu.async_copy(rows_v.at[b], acc_sh.at[dst_v.at[j0 + b]],
                         ssem[b], add=True)
    for b in range(_NBUF):
        s_desc(b, j0 + b).wait()

    @pl.when(w < _RTAIL)
    def _tail_edges():
        pltpu.async_copy(h_hbm.at[src_v.at[_RPW]], rows_v.at[0],
                         gsem[0]).wait()
        pltpu.sync_copy(rows_v.at[0], acc_sh.at[dst_v.at[_RPW]], add=True)


def _writeout(acc_sh, out_hbm, c, s):
    pltpu.sync_copy(acc_sh.at[pl.ds(s * _ZR, _ZR)],
                    out_hbm.at[c, pl.ds(s * _ZR, _ZR)])

    @pl.when(s == 0)
    def _otail():
        pltpu.sync_copy(acc_sh.at[pl.ds(_NS * _ZR, _ZTAIL)],
                        out_hbm.at[c, pl.ds(_NS * _ZR, _ZTAIL)])


def _zero_slice(z_hbm, acc_sh, s):
    pltpu.sync_copy(z_hbm, acc_sh.at[pl.ds(s * _ZR, _ZR)])

    @pl.when(s == 0)
    def _ztail():
        pltpu.sync_copy(z_hbm.at[pl.ds(0, _ZTAIL)],
                        acc_sh.at[pl.ds(_NS * _ZR, _ZTAIL)])


def _sc_segsum(h, srcr, dstr, zrows):
    @functools.partial(
        pl.kernel,
        out_type=jax.ShapeDtypeStruct((_NC, _N, _HID), jnp.float32),
        mesh=_sc_mesh(),
        scratch_types=[
            pltpu.VMEM((_RPW + 1, _CB), jnp.int32),
            pltpu.VMEM((_RPW + 1, _CB), jnp.int32),
            pltpu.VMEM((_NBUF, _CB, _HID), jnp.float32),
            pltpu.VMEM_SHARED((_N, _HID), jnp.float32),
        ] + [pltpu.SemaphoreType.DMA] * (2 * _NBUF),
        compiler_params=pltpu.CompilerParams(use_tc_tiling_on_sc=False),
    )
    def run(h_hbm, src_hbm, dst_hbm, z_hbm, out_hbm,
            src_v, dst_v, rows_v, acc_sh, *sems):
        c = lax.axis_index("c")
        s = lax.axis_index("s")
        _zero_slice(z_hbm, acc_sh, s)
        plsc.subcore_barrier()
        _segsum_body(h_hbm, src_hbm, dst_hbm, out_hbm, src_v, dst_v,
                     rows_v, acc_sh, sems[:_NBUF], sems[_NBUF:], c, s)
        plsc.subcore_barrier()
        _writeout(acc_sh, out_hbm, c, s)

    return run(h, srcr, dstr, zrows)


def _sc_segsum_deg(h, srcr, dstr, zrows, zrows8, ones_rows):
    @functools.partial(
        pl.kernel,
        out_type=(jax.ShapeDtypeStruct((_NC, _N, _HID), jnp.float32),
                  jax.ShapeDtypeStruct((_NC, _N, 8), jnp.float32)),
        mesh=_sc_mesh(),
        scratch_types=[
            pltpu.VMEM((_RPW + 1, _CB), jnp.int32),
            pltpu.VMEM((_RPW + 1, _CB), jnp.int32),
            pltpu.VMEM((_NBUF, _CB, _HID), jnp.float32),
            pltpu.VMEM((_CB, 8), jnp.float32),
            pltpu.VMEM_SHARED((_N, _HID), jnp.float32),
            pltpu.VMEM_SHARED((_N, 8), jnp.float32),
        ] + [pltpu.SemaphoreType.DMA] * (2 * _NBUF),
        compiler_params=pltpu.CompilerParams(use_tc_tiling_on_sc=False),
    )
    def run(h_hbm, src_hbm, dst_hbm, z_hbm, z8_hbm, ones_hbm,
            out_hbm, deg_hbm, src_v, dst_v, rows_v, ones_v,
            acc_sh, dacc_sh, *sems):
        c = lax.axis_index("c")
        s = lax.axis_index("s")
        w = c * _NS + s
        pltpu.sync_copy(ones_hbm, ones_v)
        _zero_slice(z_hbm, acc_sh, s)
        _zero_slice(z8_hbm, dacc_sh, s)
        plsc.subcore_barrier()
        _segsum_body(h_hbm, src_hbm, dst_hbm, out_hbm, src_v, dst_v,
                     rows_v, acc_sh, sems[:_NBUF], sems[_NBUF:], c, s)

        def dphase(j, carry):
            pltpu.sync_copy(ones_v, dacc_sh.at[dst_v.at[j]], add=True)
            return carry

        lax.fori_loop(0, _RPW, dphase, 0)

        @pl.when(w < _RTAIL)
        def _dtail():
            pltpu.sync_copy(ones_v, dacc_sh.at[dst_v.at[_RPW]], add=True)

        plsc.subcore_barrier()
        _writeout(acc_sh, out_hbm, c, s)
        _writeout(dacc_sh, deg_hbm, c, s)

    return run(h, srcr, dstr, zrows, zrows8, ones_rows)


_R2 = 1000
_N2 = _N // 2


def _tc_encoder(x2, w2, b2):
    def body(x_ref, w_ref, b_ref, o_ref):
        o_ref[...] = jnp.dot(x_ref[...], w_ref[...],
                             preferred_element_type=jnp.float32) + b_ref[...]

    return pl.pallas_call(
        body,
        grid=(_N2 // _R2,),
        in_specs=[
            pl.BlockSpec((_R2, 2 * _IN), lambda i: (i, 0)),
            pl.BlockSpec((2 * _IN, 2 * _HID), lambda i: (0, 0)),
            pl.BlockSpec((1, 2 * _HID), lambda i: (0, 0)),
        ],
        out_specs=pl.BlockSpec((_R2, 2 * _HID), lambda i: (i, 0)),
        out_shape=jax.ShapeDtypeStruct((_N2, 2 * _HID), jnp.float32),
    )(x2, w2, b2)


def _tc_inv(deg2):
    def body(d_ref, o_ref):
        inv = 1.0 / jnp.maximum(d_ref[0] + d_ref[1], 1.0)
        l_idx = lax.broadcasted_iota(jnp.int32, (128, 128), 0)
        c_idx = lax.broadcasted_iota(jnp.int32, (128, 128), 1)
        parts = []
        for m in range(8):
            sel = (l_idx == 16 * m + 8 * (c_idx // 64)).astype(jnp.float32)
            parts.append(jnp.dot(inv, sel,
                                 preferred_element_type=jnp.float32))
        o_ref[...] = jnp.stack(parts, axis=1).reshape(_N2, 128)

    return pl.pallas_call(
        body,
        grid=(1,),
        in_specs=[pl.BlockSpec((_NC, _N // 16, 128), lambda i: (0, 0, 0))],
        out_specs=pl.BlockSpec((_N2, 128), lambda i: (0, 0)),
        out_shape=jax.ShapeDtypeStruct((_N2, 128), jnp.float32),
    )(deg2)


def _tc_layer(part2, inv2, h2, w2l, w2r, scale2, bias2):
    def body(p_ref, i_ref, h_ref, wl_ref, wr_ref, s_ref, b_ref, o_ref):
        aggr = (p_ref[0] + p_ref[1]) * i_ref[...]
        y = (jnp.dot(aggr, wl_ref[...], preferred_element_type=jnp.float32)
             + jnp.dot(h_ref[...], wr_ref[...],
                       preferred_element_type=jnp.float32))
        o_ref[...] = jnp.maximum(y * s_ref[...] + b_ref[...], 0.0)

    return pl.pallas_call(
        body,
        grid=(_N2 // _R2,),
        in_specs=[
            pl.BlockSpec((_NC, _R2, 128), lambda i: (0, i, 0)),
            pl.BlockSpec((_R2, 128), lambda i: (i, 0)),
            pl.BlockSpec((_R2, 128), lambda i: (i, 0)),
            pl.BlockSpec((128, 128), lambda i: (0, 0)),
            pl.BlockSpec((128, 128), lambda i: (0, 0)),
            pl.BlockSpec((1, 128), lambda i: (0, 0)),
            pl.BlockSpec((1, 128), lambda i: (0, 0)),
        ],
        out_specs=pl.BlockSpec((_R2, 128), lambda i: (i, 0)),
        out_shape=jax.ShapeDtypeStruct((_N2, 128), jnp.float32),
    )(part2, inv2, h2, w2l, w2r, scale2, bias2)


def _tc_final(part2, inv2, h2, w2l, w2r, scale2, bias2, heads):
    (ew1t, eb1, ew2t, eb2, mw1t, mb1, mw2t, mb2, fw1t, fb1, fw2t, fb2) = heads
    steps = _N2 // _R2

    def body(p_ref, i_ref, h_ref, wl_ref, wr_ref, s_ref, b_ref,
             ew1_ref, eb1_ref, ew2_ref, eb2_ref,
             mw1_ref, mb1_ref, mw2_ref, mb2_ref,
             fw1_ref, fb1_ref, fw2_ref, fb2_ref,
             o_ref, g_ref, eth_ref, man_ref, mf_ref):
        i = pl.program_id(0)
        aggr = (p_ref[0] + p_ref[1]) * i_ref[...]
        y = (jnp.dot(aggr, wl_ref[...], preferred_element_type=jnp.float32)
             + jnp.dot(h_ref[...], wr_ref[...],
                       preferred_element_type=jnp.float32))
        y = y * s_ref[...] + b_ref[...]
        o_ref[...] = y
        blk = jnp.sum(y, axis=0, keepdims=True)

        @pl.when(i == 0)
        def _init():
            g_ref[...] = blk

        @pl.when(i > 0)
        def _acc():
            g_ref[...] = g_ref[...] + blk

        @pl.when(i == steps - 1)
        def _heads():
            gpair = g_ref[...]
            g = (gpair[:, :_OUT] + gpair[:, _OUT:]) / float(_N)
            g_ref[...] = jnp.concatenate([g, g], axis=1)
            e1 = jnp.maximum(
                jnp.dot(g, ew1_ref[...], preferred_element_type=jnp.float32)
                + eb1_ref[...], 0.0)
            eth_ref[...] = jax.nn.sigmoid(
                jnp.dot(e1, ew2_ref[...], preferred_element_type=jnp.float32)
                + eb2_ref[...])
            m1 = jnp.maximum(
                jnp.dot(g, mw1_ref[...], preferred_element_type=jnp.float32)
                + mb1_ref[...], 0.0)
            man_ref[...] = jax.nn.sigmoid(
                jnp.dot(m1, mw2_ref[...], preferred_element_type=jnp.float32)
                + mb2_ref[...])
            f1 = jnp.maximum(
                jnp.dot(g, fw1_ref[...], preferred_element_type=jnp.float32)
                + fb1_ref[...], 0.0)
            mf_ref[...] = (
                jnp.dot(f1, fw2_ref[...], preferred_element_type=jnp.float32)
                + fb2_ref[...])

    full = lambda shape: pl.BlockSpec(shape, lambda i: tuple(0 for _ in shape))
    return pl.pallas_call(
        body,
        grid=(steps,),
        in_specs=[
            pl.BlockSpec((_NC, _R2, 128), lambda i: (0, i, 0)),
            pl.BlockSpec((_R2, 128), lambda i: (i, 0)),
            pl.BlockSpec((_R2, 128), lambda i: (i, 0)),
            full((128, 2 * _OUT)),
            full((128, 2 * _OUT)),
            full((1, 2 * _OUT)),
            full((1, 2 * _OUT)),
            full((_OUT, 16)), full((1, 16)), full((16, 1)), full((1, 1)),
            full((_OUT, 16)), full((1, 16)), full((16, 1)), full((1, 1)),
            full((_OUT, 16)), full((1, 16)), full((16, 6)), full((1, 6)),
        ],
        out_specs=[
            pl.BlockSpec((_R2, 2 * _OUT), lambda i: (i, 0)),
            full((1, 2 * _OUT)),
            full((1, 1)),
            full((1, 1)),
            full((1, 6)),
        ],
        out_shape=[
            jax.ShapeDtypeStruct((_N2, 2 * _OUT), jnp.float32),
            jax.ShapeDtypeStruct((1, 2 * _OUT), jnp.float32),
            jax.ShapeDtypeStruct((1, 1), jnp.float32),
            jax.ShapeDtypeStruct((1, 1), jnp.float32),
            jax.ShapeDtypeStruct((1, 6), jnp.float32),
        ],
    )(part2, inv2, h2, w2l, w2r, scale2, bias2,
      ew1t, eb1, ew2t, eb2, mw1t, mb1, mw2t, mb2, fw1t, fb1, fw2t, fb2)


def _bdiag(w):
    i, o = w.shape
    z = jnp.zeros((2 * i, 2 * o), w.dtype)
    return z.at[:i, :o].set(w).at[i:, o:].set(w)


def _dup(v):
    return jnp.concatenate([v, v], axis=1)


def kernel(x, edge_index, params):
    p = params
    srcr = edge_index[0].reshape(_ROWS, _CB)
    dstr = edge_index[1].reshape(_ROWS, _CB)
    zrows = jnp.zeros((_ZR, _HID), jnp.float32)
    zrows8 = jnp.zeros((_ZR, 8), jnp.float32)
    ones_rows = jnp.ones((_CB, 8), jnp.float32)

    x2 = x.reshape(_N2, 2 * _IN)
    h2 = _tc_encoder(x2, _bdiag(p['enc_W'].T),
                     _dup(p['enc_b'].reshape(1, -1)))

    inv2 = None
    for i in range(3):
        lp = p['sage'][i]
        bn = p['bn'][i]
        scale = (bn['gamma'] / jnp.sqrt(bn['var'] + 1e-5)).reshape(1, -1)
        bias = ((lp['bl'] - bn['mean']).reshape(1, -1) * scale
                + bn['beta'].reshape(1, -1))
        h_flat = h2.reshape(_N, _HID)
        if i == 0:
            part, degp = _sc_segsum_deg(h_flat, srcr, dstr,
                                        zrows, zrows8, ones_rows)
            inv2 = _tc_inv(degp.reshape(_NC, _N // 16, 128))
        else:
            part = _sc_segsum(h_flat, srcr, dstr, zrows)
        part2 = part.reshape(_NC, _N2, 128)
        w2l = _bdiag(lp['Wl'].T)
        w2r = _bdiag(lp['Wr'].T)
        if i < 2:
            h2 = _tc_layer(part2, inv2, h2, w2l, w2r,
                           _dup(scale), _dup(bias))
        else:
            heads = (
                p['eth_W1'].T, p['eth_b1'].reshape(1, -1),
                p['eth_W2'].T, p['eth_b2'].reshape(1, -1),
                p['man_W1'].T, p['man_b1'].reshape(1, -1),
                p['man_W2'].T, p['man_b2'].reshape(1, -1),
                p['mf_W1'].T, p['mf_b1'].reshape(1, -1),
                p['mf_W2'].T, p['mf_b2'].reshape(1, -1),
            )
            emb2, gpair, eth, man, mf = _tc_final(
                part2, inv2, h2, w2l, w2r, _dup(scale), _dup(bias), heads)
    node_emb = emb2.reshape(_N, _OUT)
    g = gpair[:, :_OUT]
    return (node_emb, g, eth, man, mf)


# speedup vs baseline: 17.1474x; 1.0434x over previous
import functools

import jax
import jax.numpy as jnp
from jax import lax
from jax.experimental import pallas as pl
from jax.experimental.pallas import tpu as pltpu
from jax.experimental.pallas import tpu_sc as plsc

_N = 10000
_E = 320000
_IN = 128
_HID = 64
_OUT = 32

_NC = 2
_NS = 16
_NW = _NC * _NS
_CB = 128
_ROWS = _E // _CB
_RPW = _ROWS // _NW
_RTAIL = _ROWS - _RPW * _NW
_ZR = 624
_ZTAIL = _N - _NS * _ZR
_NBUF = 3


def _sc_mesh():
    return plsc.VectorSubcoreMesh(
        core_axis_name="c", subcore_axis_name="s",
        num_cores=_NC, num_subcores=_NS)


def _segsum_body(h_hbm, edges_hbm, out_hbm,
                 src_v, dst_v, rows_v, acc_sh, gsem, ssem, c, s):
    w = c * _NS + s
    base = w * _RPW
    pltpu.sync_copy(edges_hbm.at[0, pl.ds(base, _RPW)],
                    src_v.at[pl.ds(0, _RPW)])
    pltpu.sync_copy(edges_hbm.at[1, pl.ds(base, _RPW)],
                    dst_v.at[pl.ds(0, _RPW)])

    @pl.when(w < _RTAIL)
    def _tail_idx():
        pltpu.sync_copy(edges_hbm.at[0, pl.ds(_RPW * _NW + w, 1)],
                        src_v.at[pl.ds(_RPW, 1)])
        pltpu.sync_copy(edges_hbm.at[1, pl.ds(_RPW * _NW + w, 1)],
                        dst_v.at[pl.ds(_RPW, 1)])

    def g_desc(b, j):
        return pltpu.make_async_copy(
            h_hbm.at[src_v.at[j]], rows_v.at[b], gsem[b])

    def s_desc(b, j):
        return pltpu.make_async_copy(
            rows_v.at[b], acc_sh.at[dst_v.at[j]], ssem[b])

    for b in range(_NBUF):
        g_desc(b, b).start()

    def step(k, carry):
        j0 = k * _NBUF
        for b in range(_NBUF):
            g_desc(b, j0 + b).wait()
            pltpu.async_copy(rows_v.at[b], acc_sh.at[dst_v.at[j0 + b]],
                             ssem[b], add=True)
        for b in range(_NBUF):
            s_desc(b, j0 + b).wait()
            g_desc(b, j0 + _NBUF + b).start()
        return carry

    lax.fori_loop(0, _RPW // _NBUF - 1, step, 0)

    j0 = _RPW - _NBUF
    for b in range(_NBUF):
        g_desc(b, j0 + b).wait()
        pltpu.async_copy(rows_v.at[b], acc_sh.at[dst_v.at[j0 + b]],
                         ssem[b], add=True)
    for b in range(_NBUF):
        s_desc(b, j0 + b).wait()

    @pl.when(w < _RTAIL)
    def _tail_edges():
        pltpu.async_copy(h_hbm.at[src_v.at[_RPW]], rows_v.at[0],
                         gsem[0]).wait()
        pltpu.sync_copy(rows_v.at[0], acc_sh.at[dst_v.at[_RPW]], add=True)


def _writeout(acc_sh, out_hbm, c, s):
    pltpu.sync_copy(acc_sh.at[pl.ds(s * _ZR, _ZR)],
                    out_hbm.at[c, pl.ds(s * _ZR, _ZR)])

    @pl.when(s == 0)
    def _otail():
        pltpu.sync_copy(acc_sh.at[pl.ds(_NS * _ZR, _ZTAIL)],
                        out_hbm.at[c, pl.ds(_NS * _ZR, _ZTAIL)])


def _zero_slice(z_hbm, acc_sh, s):
    pltpu.sync_copy(z_hbm, acc_sh.at[pl.ds(s * _ZR, _ZR)])

    @pl.when(s == 0)
    def _ztail():
        pltpu.sync_copy(z_hbm.at[pl.ds(0, _ZTAIL)],
                        acc_sh.at[pl.ds(_NS * _ZR, _ZTAIL)])


def _sc_segsum(h, edges, zrows):
    @functools.partial(
        pl.kernel,
        out_type=jax.ShapeDtypeStruct((_NC, _N, _HID), jnp.float32),
        mesh=_sc_mesh(),
        scratch_types=[
            pltpu.VMEM((_RPW + 1, _CB), jnp.int32),
            pltpu.VMEM((_RPW + 1, _CB), jnp.int32),
            pltpu.VMEM((_NBUF, _CB, _HID), jnp.float32),
            pltpu.VMEM_SHARED((_N, _HID), jnp.float32),
        ] + [pltpu.SemaphoreType.DMA] * (2 * _NBUF),
        compiler_params=pltpu.CompilerParams(use_tc_tiling_on_sc=False),
    )
    def run(h_hbm, edges_hbm, z_hbm, out_hbm,
            src_v, dst_v, rows_v, acc_sh, *sems):
        c = lax.axis_index("c")
        s = lax.axis_index("s")
        _zero_slice(z_hbm, acc_sh, s)
        plsc.subcore_barrier()
        _segsum_body(h_hbm, edges_hbm, out_hbm, src_v, dst_v,
                     rows_v, acc_sh, sems[:_NBUF], sems[_NBUF:], c, s)
        plsc.subcore_barrier()
        _writeout(acc_sh, out_hbm, c, s)

    return run(h, edges, zrows)


def _sc_segsum_deg(h, edges, zrows, zrows8, ones_rows):
    @functools.partial(
        pl.kernel,
        out_type=(jax.ShapeDtypeStruct((_NC, _N, _HID), jnp.float32),
                  jax.ShapeDtypeStruct((_NC, _N, 8), jnp.float32)),
        mesh=_sc_mesh(),
        scratch_types=[
            pltpu.VMEM((_RPW + 1, _CB), jnp.int32),
            pltpu.VMEM((_RPW + 1, _CB), jnp.int32),
            pltpu.VMEM((_NBUF, _CB, _HID), jnp.float32),
            pltpu.VMEM((_CB, 8), jnp.float32),
            pltpu.VMEM_SHARED((_N, _HID), jnp.float32),
            pltpu.VMEM_SHARED((_N, 8), jnp.float32),
        ] + [pltpu.SemaphoreType.DMA] * (2 * _NBUF),
        compiler_params=pltpu.CompilerParams(use_tc_tiling_on_sc=False),
    )
    def run(h_hbm, edges_hbm, z_hbm, z8_hbm, ones_hbm,
            out_hbm, deg_hbm, src_v, dst_v, rows_v, ones_v,
            acc_sh, dacc_sh, *sems):
        c = lax.axis_index("c")
        s = lax.axis_index("s")
        w = c * _NS + s
        pltpu.sync_copy(ones_hbm, ones_v)
        _zero_slice(z_hbm, acc_sh, s)
        _zero_slice(z8_hbm, dacc_sh, s)
        plsc.subcore_barrier()
        _segsum_body(h_hbm, edges_hbm, out_hbm, src_v, dst_v,
                     rows_v, acc_sh, sems[:_NBUF], sems[_NBUF:], c, s)

        def dphase(j, carry):
            pltpu.sync_copy(ones_v, dacc_sh.at[dst_v.at[j]], add=True)
            return carry

        lax.fori_loop(0, _RPW, dphase, 0)

        @pl.when(w < _RTAIL)
        def _dtail():
            pltpu.sync_copy(ones_v, dacc_sh.at[dst_v.at[_RPW]], add=True)

        plsc.subcore_barrier()
        _writeout(acc_sh, out_hbm, c, s)
        _writeout(dacc_sh, deg_hbm, c, s)

    return run(h, edges, zrows, zrows8, ones_rows)


_R2 = 1000
_N2 = _N // 2


def _pair_dot(x2, w_ref, half):
    w = w_ref[...]
    ya = jnp.dot(x2[:, :half], w, preferred_element_type=jnp.float32)
    yb = jnp.dot(x2[:, half:], w, preferred_element_type=jnp.float32)
    return jnp.concatenate([ya, yb], axis=1)


def _tc_encoder(x2, wt, b):
    def body(x_ref, w_ref, b_ref, o_ref):
        bb = jnp.concatenate([b_ref[...], b_ref[...]], axis=1)
        o_ref[...] = _pair_dot(x_ref[...], w_ref, _IN) + bb

    return pl.pallas_call(
        body,
        grid=(_N2 // _R2,),
        in_specs=[
            pl.BlockSpec((_R2, 2 * _IN), lambda i: (i, 0)),
            pl.BlockSpec((_IN, _HID), lambda i: (0, 0)),
            pl.BlockSpec((1, _HID), lambda i: (0, 0)),
        ],
        out_specs=pl.BlockSpec((_R2, 2 * _HID), lambda i: (i, 0)),
        out_shape=jax.ShapeDtypeStruct((_N2, 2 * _HID), jnp.float32),
    )(x2, wt, b)


def _tc_inv(deg2):
    def body(d_ref, o_ref):
        inv = 1.0 / jnp.maximum(d_ref[0] + d_ref[1], 1.0)
        l_idx = lax.broadcasted_iota(jnp.int32, (128, 128), 0)
        c_idx = lax.broadcasted_iota(jnp.int32, (128, 128), 1)
        parts = []
        for m in range(8):
            sel = (l_idx == 16 * m + 8 * (c_idx // 64)).astype(jnp.float32)
            parts.append(jnp.dot(inv, sel,
                                 preferred_element_type=jnp.float32))
        o_ref[...] = jnp.stack(parts, axis=1).reshape(_N2, 128)

    return pl.pallas_call(
        body,
        grid=(1,),
        in_specs=[pl.BlockSpec((_NC, _N // 16, 128), lambda i: (0, 0, 0))],
        out_specs=pl.BlockSpec((_N2, 128), lambda i: (0, 0)),
        out_shape=jax.ShapeDtypeStruct((_N2, 128), jnp.float32),
    )(deg2)


def _tc_layer(part2, inv2, h2, wlt, wrt, scale, bias):
    def body(p_ref, i_ref, h_ref, wl_ref, wr_ref, s_ref, b_ref, o_ref):
        aggr = (p_ref[0] + p_ref[1]) * i_ref[...]
        y = _pair_dot(aggr, wl_ref, _HID) + _pair_dot(h_ref[...], wr_ref, _HID)
        ss = jnp.concatenate([s_ref[...], s_ref[...]], axis=1)
        bb = jnp.concatenate([b_ref[...], b_ref[...]], axis=1)
        o_ref[...] = jnp.maximum(y * ss + bb, 0.0)

    return pl.pallas_call(
        body,
        grid=(_N2 // _R2,),
        in_specs=[
            pl.BlockSpec((_NC, _R2, 128), lambda i: (0, i, 0)),
            pl.BlockSpec((_R2, 128), lambda i: (i, 0)),
            pl.BlockSpec((_R2, 128), lambda i: (i, 0)),
            pl.BlockSpec((_HID, _HID), lambda i: (0, 0)),
            pl.BlockSpec((_HID, _HID), lambda i: (0, 0)),
            pl.BlockSpec((1, _HID), lambda i: (0, 0)),
            pl.BlockSpec((1, _HID), lambda i: (0, 0)),
        ],
        out_specs=pl.BlockSpec((_R2, 128), lambda i: (i, 0)),
        out_shape=jax.ShapeDtypeStruct((_N2, 128), jnp.float32),
    )(part2, inv2, h2, wlt, wrt, scale, bias)


def _tc_final(part2, inv2, h2, wlt, wrt, scale, bias, heads):
    (ew1t, eb1, ew2t, eb2, mw1t, mb1, mw2t, mb2, fw1t, fb1, fw2t, fb2) = heads
    steps = _N2 // _R2

    def body(p_ref, i_ref, h_ref, wl_ref, wr_ref, s_ref, b_ref,
             ew1_ref, eb1_ref, ew2_ref, eb2_ref,
             mw1_ref, mb1_ref, mw2_ref, mb2_ref,
             fw1_ref, fb1_ref, fw2_ref, fb2_ref,
             o_ref, g_ref, eth_ref, man_ref, mf_ref):
        i = pl.program_id(0)
        aggr = (p_ref[0] + p_ref[1]) * i_ref[...]
        y = _pair_dot(aggr, wl_ref, _HID) + _pair_dot(h_ref[...], wr_ref, _HID)
        ss = jnp.concatenate([s_ref[...], s_ref[...]], axis=1)
        bb = jnp.concatenate([b_ref[...], b_ref[...]], axis=1)
        y = y * ss + bb
        o_ref[...] = y
        blk = jnp.sum(y, axis=0, keepdims=True)

        @pl.when(i == 0)
        def _init():
            g_ref[...] = blk

        @pl.when(i > 0)
        def _acc():
            g_ref[...] = g_ref[...] + blk

        @pl.when(i == steps - 1)
        def _heads():
            gpair = g_ref[...]
            g = (gpair[:, :_OUT] + gpair[:, _OUT:]) / float(_N)
            g_ref[...] = jnp.concatenate([g, g], axis=1)
            e1 = jnp.maximum(
                jnp.dot(g, ew1_ref[...], preferred_element_type=jnp.float32)
                + eb1_ref[...], 0.0)
            eth_ref[...] = jax.nn.sigmoid(
                jnp.dot(e1, ew2_ref[...], preferred_element_type=jnp.float32)
                + eb2_ref[...])
            m1 = jnp.maximum(
                jnp.dot(g, mw1_ref[...], preferred_element_type=jnp.float32)
                + mb1_ref[...], 0.0)
            man_ref[...] = jax.nn.sigmoid(
                jnp.dot(m1, mw2_ref[...], preferred_element_type=jnp.float32)
                + mb2_ref[...])
            f1 = jnp.maximum(
                jnp.dot(g, fw1_ref[...], preferred_element_type=jnp.float32)
                + fb1_ref[...], 0.0)
            mf_ref[...] = (
                jnp.dot(f1, fw2_ref[...], preferred_element_type=jnp.float32)
                + fb2_ref[...])

    full = lambda shape: pl.BlockSpec(shape, lambda i: tuple(0 for _ in shape))
    return pl.pallas_call(
        body,
        grid=(steps,),
        in_specs=[
            pl.BlockSpec((_NC, _R2, 128), lambda i: (0, i, 0)),
            pl.BlockSpec((_R2, 128), lambda i: (i, 0)),
            pl.BlockSpec((_R2, 128), lambda i: (i, 0)),
            full((_HID, _OUT)),
            full((_HID, _OUT)),
            full((1, _OUT)),
            full((1, _OUT)),
            full((_OUT, 16)), full((1, 16)), full((16, 1)), full((1, 1)),
            full((_OUT, 16)), full((1, 16)), full((16, 1)), full((1, 1)),
            full((_OUT, 16)), full((1, 16)), full((16, 6)), full((1, 6)),
        ],
        out_specs=[
            pl.BlockSpec((_R2, 2 * _OUT), lambda i: (i, 0)),
            full((1, 2 * _OUT)),
            full((1, 1)),
            full((1, 1)),
            full((1, 6)),
        ],
        out_shape=[
            jax.ShapeDtypeStruct((_N2, 2 * _OUT), jnp.float32),
            jax.ShapeDtypeStruct((1, 2 * _OUT), jnp.float32),
            jax.ShapeDtypeStruct((1, 1), jnp.float32),
            jax.ShapeDtypeStruct((1, 1), jnp.float32),
            jax.ShapeDtypeStruct((1, 6), jnp.float32),
        ],
    )(part2, inv2, h2, wlt, wrt, scale, bias,
      ew1t, eb1, ew2t, eb2, mw1t, mb1, mw2t, mb2, fw1t, fb1, fw2t, fb2)


def kernel(x, edge_index, params):
    p = params
    edges = edge_index.reshape(2, _ROWS, _CB)
    zrows = jnp.zeros((_ZR, _HID), jnp.float32)
    zrows8 = jnp.zeros((_ZR, 8), jnp.float32)
    ones_rows = jnp.ones((_CB, 8), jnp.float32)

    x2 = x.reshape(_N2, 2 * _IN)
    h2 = _tc_encoder(x2, p['enc_W'].T, p['enc_b'].reshape(1, -1))

    inv2 = None
    for i in range(3):
        lp = p['sage'][i]
        bn = p['bn'][i]
        scale = (bn['gamma'] / jnp.sqrt(bn['var'] + 1e-5)).reshape(1, -1)
        bias = ((lp['bl'] - bn['mean']).reshape(1, -1) * scale
                + bn['beta'].reshape(1, -1))
        h_flat = h2.reshape(_N, _HID)
        if i == 0:
            part, degp = _sc_segsum_deg(h_flat, edges,
                                        zrows, zrows8, ones_rows)
            inv2 = _tc_inv(degp.reshape(_NC, _N // 16, 128))
        else:
            part = _sc_segsum(h_flat, edges, zrows)
        part2 = part.reshape(_NC, _N2, 128)
        if i < 2:
            h2 = _tc_layer(part2, inv2, h2, lp['Wl'].T, lp['Wr'].T,
                           scale, bias)
        else:
            heads = (
                p['eth_W1'].T, p['eth_b1'].reshape(1, -1),
                p['eth_W2'].T, p['eth_b2'].reshape(1, -1),
                p['man_W1'].T, p['man_b1'].reshape(1, -1),
                p['man_W2'].T, p['man_b2'].reshape(1, -1),
                p['mf_W1'].T, p['mf_b1'].reshape(1, -1),
                p['mf_W2'].T, p['mf_b2'].reshape(1, -1),
            )
            emb2, gpair, eth, man, mf = _tc_final(
                part2, inv2, h2, lp['Wl'].T, lp['Wr'].T, scale, bias, heads)
    node_emb = emb2.reshape(_N, _OUT)
    g = gpair[:, :_OUT]
    return (node_emb, g, eth, man, mf)
